# baseline (speedup 1.0000x reference)
import functools

import jax
import jax.numpy as jnp
from jax import lax
from jax.experimental import pallas as pl
from jax.experimental.pallas import tpu as pltpu
from jax.experimental.pallas import tpu_sc as plsc

_NUM_NODES = 1000000
_EMBED_DIM = 64
_BATCH = 16384

_INFO = plsc.get_sparse_core_info()
_NC = _INFO.num_cores
_NS = _INFO.num_subcores
_NW = _NC * _NS
_B_PER_W = _BATCH // _NW
_NBUF = 8
_NGRP = _B_PER_W // 16


@functools.partial(
    pl.kernel,
    mesh=plsc.VectorSubcoreMesh(core_axis_name="c", subcore_axis_name="s"),
    out_type=jax.ShapeDtypeStruct((_BATCH * _EMBED_DIM,), jnp.float32),
    scratch_types=[
        pltpu.VMEM((_B_PER_W + 16,), jnp.int32),
        pltpu.VMEM((_B_PER_W * _EMBED_DIM,), jnp.float32),
    ]
    + [pltpu.VMEM((_EMBED_DIM, 128), jnp.float32) for _ in range(_NBUF)]
    + [pltpu.SemaphoreType.DMA for _ in range(_NBUF)],
    compiler_params=pltpu.CompilerParams(needs_layout_passes=False),
)
def _gather_kernel(idx_hbm, wt_hbm, out_hbm, idx_v, out_v, *blocks_and_sems):
    blocks = blocks_and_sems[:_NBUF]
    sems = blocks_and_sems[_NBUF:]
    wid = lax.axis_index("s") * _NC + lax.axis_index("c")
    base = wid * _B_PER_W
    pltpu.sync_copy(idx_hbm.at[pl.ds(base, _B_PER_W)],
                    idx_v.at[pl.ds(0, _B_PER_W)])

    rows = [lax.iota(jnp.int32, 16) + 16 * k for k in range(4)]

    def tcol_of(i):
        return pl.multiple_of(
            lax.shift_left(lax.shift_right_logical(i, 7), 7), 128)

    def fire(i, b):
        t = tcol_of(i)
        for bb in range(8):
            pltpu.make_async_copy(
                wt_hbm.at[pl.ds(8 * bb, 8), pl.ds(t, 128)],
                blocks[b].at[pl.ds(8 * bb, 8), :], sems[b]).start()

    iv0 = idx_v[pl.ds(0, 16)]
    for b in range(_NBUF):
        fire(iv0[b], b)

    def group(g, _):
        jo = g * 16
        iv = idx_v[pl.ds(jo, 16)]
        ivn = idx_v[pl.ds(jo + _NBUF, 16)]
        for k in range(16):
            b = k % _NBUF
            j = jo + k
            pltpu.make_async_copy(
                wt_hbm.at[:, pl.ds(0, 128)], blocks[b], sems[b]).wait()
            lane = jnp.full((16,), iv[k] & 127, jnp.int32)
            for r in range(4):
                v = plsc.load_gather(blocks[b], [rows[r], lane])
                out_v[pl.ds(j * _EMBED_DIM + 16 * r, 16)] = v

            @pl.when(j < _B_PER_W - _NBUF)
            def _():
                fire(ivn[k], b)

        return _

    lax.fori_loop(0, _NGRP, group, None)
    pltpu.sync_copy(out_v, out_hbm.at[pl.ds(base * _EMBED_DIM,
                                            _B_PER_W * _EMBED_DIM)])


def kernel(indices, weight):
    flat = _gather_kernel(indices.astype(jnp.int32), weight.T)
    return flat.reshape(_BATCH, _EMBED_DIM)


# speedup vs baseline: 1.0516x; 1.0516x over previous
import functools

import jax
import jax.numpy as jnp
from jax import lax
from jax.experimental import pallas as pl
from jax.experimental.pallas import tpu as pltpu
from jax.experimental.pallas import tpu_sc as plsc

_NUM_NODES = 1000000
_EMBED_DIM = 64
_BATCH = 16384

_INFO = plsc.get_sparse_core_info()
_NC = _INFO.num_cores
_NS = _INFO.num_subcores
_NW = _NC * _NS
_B_PER_W = _BATCH // _NW
_NBUF = 8
_NWIN = _B_PER_W // 128


@functools.partial(
    pl.kernel,
    mesh=plsc.VectorSubcoreMesh(core_axis_name="c", subcore_axis_name="s"),
    out_type=jax.ShapeDtypeStruct((_EMBED_DIM, _BATCH), jnp.float32),
    scratch_types=[
        pltpu.VMEM((_B_PER_W + 16,), jnp.int32),
        pltpu.VMEM((_EMBED_DIM, 128), jnp.float32),
        pltpu.VMEM((_EMBED_DIM, 128), jnp.float32),
    ]
    + [pltpu.VMEM((_EMBED_DIM, 128), jnp.float32) for _ in range(_NBUF)]
    + [pltpu.SemaphoreType.DMA for _ in range(_NBUF)]
    + [pltpu.SemaphoreType.DMA],
    compiler_params=pltpu.CompilerParams(needs_layout_passes=False),
)
def _gather_kernel(idx_hbm, wt_hbm, out_hbm, idx_v, stag_a, stag_b, *rest):
    blocks = rest[:_NBUF]
    sems = rest[_NBUF:2 * _NBUF]
    sem_o = rest[2 * _NBUF]
    stags = (stag_a, stag_b)
    wid = lax.axis_index("s") * _NC + lax.axis_index("c")
    base = wid * _B_PER_W
    pltpu.sync_copy(idx_hbm.at[pl.ds(base, _B_PER_W)],
                    idx_v.at[pl.ds(0, _B_PER_W)])

    rows = [lax.iota(jnp.int32, 16) + 16 * k for k in range(4)]

    def tcol_of(i):
        return pl.multiple_of(
            lax.shift_left(lax.shift_right_logical(i, 7), 7), 128)

    def fire(i, b):
        pltpu.make_async_copy(
            wt_hbm.at[:, pl.ds(tcol_of(i), 128)], blocks[b], sems[b]).start()

    iv0 = idx_v[pl.ds(0, 16)]
    for b in range(_NBUF):
        fire(iv0[b], b)

    for win in range(_NWIN):
        stag = stags[win % 2]
        if win >= 2:
            pltpu.make_async_copy(
                wt_hbm.at[:, pl.ds(0, 128)], stag, sem_o).wait()

        def group(g, _):
            jo = win * 128 + g * 16
            iv = idx_v[pl.ds(jo, 16)]
            ivn = idx_v[pl.ds(jo + _NBUF, 16)]
            for k in range(16):
                b = k % _NBUF
                j = jo + k
                pltpu.make_async_copy(
                    wt_hbm.at[:, pl.ds(0, 128)], blocks[b], sems[b]).wait()
                lane = jnp.full((16,), iv[k] & 127, jnp.int32)
                colj = jnp.full((16,), g * 16 + k, jnp.int32)
                for r in range(4):
                    v = plsc.load_gather(blocks[b], [rows[r], lane])
                    plsc.store_scatter(stag, [rows[r], colj], v)

                @pl.when(j < _B_PER_W - _NBUF)
                def _():
                    fire(ivn[k], b)

            return _

        lax.fori_loop(0, 8, group, None)
        pltpu.make_async_copy(
            stag, out_hbm.at[:, pl.ds(base + win * 128, 128)],
            sem_o).start()

    for s in stags:
        pltpu.make_async_copy(
            wt_hbm.at[:, pl.ds(0, 128)], s, sem_o).wait()


def kernel(indices, weight):
    out_t = _gather_kernel(indices.astype(jnp.int32), weight.T)
    return out_t.T
